# Initial kernel scaffold; baseline (speedup 1.0000x reference)
import jax
import jax.numpy as jnp
from jax.experimental import pallas as pl


def kernel(X, edge_index, W, b, gamma, beta):
    raise NotImplementedError("write your pallas kernel here")


# speedup vs baseline: 16.1763x; 16.1763x over previous
import jax
import jax.numpy as jnp
from jax import lax
from jax.experimental import pallas as pl
from jax.experimental.pallas import tpu as pltpu
from jax.experimental.pallas import tpu_sc as plsc

N = 10000
E = 160000
C = 256
HALF = C // 2
EPS = 1e-5

NC = 2
NS = 16

ROW_BLK = 1000
GRID = N // ROW_BLK

EK = 125
EROWS = E // EK
DEG_RPW = EROWS // (NC * NS)
AGG_RPT = EROWS // NS
AGG_BATCH = 16

NPAD = 10240
STRIPE = NPAD // NS
SEED_CHUNK = 64


def _sc_mesh():
    return plsc.VectorSubcoreMesh(
        core_axis_name="c", subcore_axis_name="s", num_cores=NC, num_subcores=NS
    )


def _tc1_body(x_ref, wt_ref, b_ref, h_ref, stats_ref):
    i = pl.program_id(0)
    h = jnp.dot(x_ref[...], wt_ref[...], preferred_element_type=jnp.float32)
    h = h + b_ref[...]
    h_ref[...] = h
    s = jnp.sum(h, axis=0, keepdims=True)
    ss = jnp.sum(h * h, axis=0, keepdims=True)
    blk = jnp.concatenate([s, ss, jnp.zeros((6, C), jnp.float32)], axis=0)
    prev = jnp.where(i == 0, jnp.zeros_like(blk), stats_ref[...])
    stats_ref[...] = prev + blk


def _tc1(x, wt, b2):
    return pl.pallas_call(
        _tc1_body,
        grid=(GRID,),
        in_specs=[
            pl.BlockSpec((ROW_BLK, C), lambda i: (i, 0)),
            pl.BlockSpec((C, C), lambda i: (0, 0)),
            pl.BlockSpec((1, C), lambda i: (0, 0)),
        ],
        out_specs=[
            pl.BlockSpec((ROW_BLK, C), lambda i: (i, 0)),
            pl.BlockSpec((8, C), lambda i: (0, 0)),
        ],
        out_shape=[
            jax.ShapeDtypeStruct((N, C), jnp.float32),
            jax.ShapeDtypeStruct((8, C), jnp.float32),
        ],
    )(x, wt, b2)


def _deg_body(dst_hbm, deg_hbm, idx_v, ones_v, stage_v, deg_sh):
    c = lax.axis_index("c")
    s = lax.axis_index("s")
    w = c * NS + s

    one16 = jnp.ones((16,), jnp.float32)
    for k in range(8):
        ones_v[pl.ds(k * 16, 16)] = one16
    z16 = jnp.zeros((16,), jnp.float32)
    for k in range(STRIPE // 16):
        stage_v[pl.ds(k * 16, 16)] = z16

    off = pl.multiple_of(s * STRIPE, 8)
    pltpu.sync_copy(stage_v, deg_sh.at[pl.ds(off, STRIPE)])
    plsc.subcore_barrier()

    woff = pl.multiple_of(w * DEG_RPW, 8)
    pltpu.sync_copy(dst_hbm.at[pl.ds(woff, DEG_RPW)], idx_v)

    def body(j, carry):
        pltpu.sync_copy(ones_v.at[pl.ds(0, EK)],
                        deg_sh.at[idx_v.at[j]], add=True)
        return carry

    lax.fori_loop(0, DEG_RPW, body, 0)
    plsc.subcore_barrier()

    pltpu.sync_copy(deg_sh.at[pl.ds(off, STRIPE)], stage_v)
    goff = pl.multiple_of(c * NPAD + s * STRIPE, 8)
    pltpu.sync_copy(stage_v, deg_hbm.at[pl.ds(goff, STRIPE)])


def _sc_degree(dst_deg):
    return pl.kernel(
        _deg_body,
        out_type=jax.ShapeDtypeStruct((NC * NPAD,), jnp.float32),
        mesh=_sc_mesh(),
        scratch_types=[
            pltpu.VMEM((DEG_RPW, EK), jnp.int32),
            pltpu.VMEM((128,), jnp.float32),
            pltpu.VMEM((STRIPE,), jnp.float32),
            pltpu.VMEM_SHARED((NPAD,), jnp.float32),
        ],
    )(dst_deg)


def _tc2_body(h_ref, stats_ref, p0_ref, p1_ref, g_ref, be_ref, hs2_ref):
    inv_n = 1.0 / N
    mean = stats_ref[0:1, :] * inv_n
    var = stats_ref[1:2, :] * inv_n - mean * mean
    a = g_ref[...] * lax.rsqrt(var + EPS)
    cc = be_ref[...] - a * mean
    dinv = lax.rsqrt(p0_ref[...] + p1_ref[...] + 1.0)
    hs = (h_ref[...] * a + cc) * dinv
    hs2_ref[...] = jnp.stack([hs[:, :HALF], hs[:, HALF:]], axis=0)


def _tc2(h, stats, p0, p1, g2, be2):
    return pl.pallas_call(
        _tc2_body,
        grid=(GRID,),
        in_specs=[
            pl.BlockSpec((ROW_BLK, C), lambda i: (i, 0)),
            pl.BlockSpec((8, C), lambda i: (0, 0)),
            pl.BlockSpec((ROW_BLK, 1), lambda i: (i, 0)),
            pl.BlockSpec((ROW_BLK, 1), lambda i: (i, 0)),
            pl.BlockSpec((1, C), lambda i: (0, 0)),
            pl.BlockSpec((1, C), lambda i: (0, 0)),
        ],
        out_specs=pl.BlockSpec((NC, ROW_BLK, HALF), lambda i: (0, i, 0)),
        out_shape=jax.ShapeDtypeStruct((NC, NPAD, HALF), jnp.float32),
    )(h, stats, p0, p1, g2, be2)


def _agg_body(hs2_hbm, src_hbm, dst_hbm, agg2_hbm,
              idx_s, idx_d, rows, stage, agg_sh):
    c = lax.axis_index("c")
    s = lax.axis_index("s")
    off = pl.multiple_of(s * STRIPE, 8)
    hs_ref = hs2_hbm.at[c]
    out_ref = agg2_hbm.at[c]

    for k in range(STRIPE // SEED_CHUNK):
        koff = pl.multiple_of(off + k * SEED_CHUNK, 8)
        pltpu.sync_copy(hs_ref.at[pl.ds(koff, SEED_CHUNK)], stage)
        pltpu.sync_copy(stage, agg_sh.at[pl.ds(koff, SEED_CHUNK)])
    plsc.subcore_barrier()

    for bt in range(AGG_RPT // AGG_BATCH):
        boff = pl.multiple_of(s * AGG_RPT + bt * AGG_BATCH, 8)
        pltpu.sync_copy(src_hbm.at[pl.ds(boff, AGG_BATCH)], idx_s)
        pltpu.sync_copy(dst_hbm.at[pl.ds(boff, AGG_BATCH)], idx_d)

        def body(j, carry):
            pltpu.sync_copy(hs_ref.at[idx_s.at[j]], rows)
            pltpu.sync_copy(rows, agg_sh.at[idx_d.at[j]], add=True)
            return carry

        lax.fori_loop(0, AGG_BATCH, body, 0)
    plsc.subcore_barrier()

    for k in range(STRIPE // SEED_CHUNK):
        koff = pl.multiple_of(off + k * SEED_CHUNK, 8)
        pltpu.sync_copy(agg_sh.at[pl.ds(koff, SEED_CHUNK)], stage)
        pltpu.sync_copy(stage, out_ref.at[pl.ds(koff, SEED_CHUNK)])


def _sc_agg(hs2, src_agg, dst_agg):
    return pl.kernel(
        _agg_body,
        out_type=jax.ShapeDtypeStruct((NC, NPAD, HALF), jnp.float32),
        mesh=_sc_mesh(),
        scratch_types=[
            pltpu.VMEM((AGG_BATCH, EK), jnp.int32),
            pltpu.VMEM((AGG_BATCH, EK), jnp.int32),
            pltpu.VMEM((EK, HALF), jnp.float32),
            pltpu.VMEM((SEED_CHUNK, HALF), jnp.float32),
            pltpu.VMEM_SHARED((NPAD, HALF), jnp.float32),
        ],
    )(hs2, src_agg, dst_agg)


def _tc3_body(a2_ref, p0_ref, p1_ref, out_ref):
    dinv = lax.rsqrt(p0_ref[...] + p1_ref[...] + 1.0)
    a2 = a2_ref[...]
    o = jnp.concatenate([a2[0], a2[1]], axis=1) * dinv
    out_ref[...] = jnp.maximum(o, 0.0)


def _tc3(agg2, p0, p1):
    return pl.pallas_call(
        _tc3_body,
        grid=(GRID,),
        in_specs=[
            pl.BlockSpec((NC, ROW_BLK, HALF), lambda i: (0, i, 0)),
            pl.BlockSpec((ROW_BLK, 1), lambda i: (i, 0)),
            pl.BlockSpec((ROW_BLK, 1), lambda i: (i, 0)),
        ],
        out_specs=pl.BlockSpec((ROW_BLK, C), lambda i: (i, 0)),
        out_shape=jax.ShapeDtypeStruct((N, C), jnp.float32),
    )(agg2, p0, p1)


def kernel(X, edge_index, W, b, gamma, beta):
    src = edge_index[0]
    dst = edge_index[1]
    src2d = src.reshape(EROWS, EK)
    dst2d = dst.reshape(EROWS, EK)
    wt = W.T
    b2 = b.reshape(1, C)
    g2 = gamma.reshape(1, C)
    be2 = beta.reshape(1, C)

    h, stats = _tc1(X, wt, b2)
    deg2 = _sc_degree(dst2d)
    p0 = deg2[:NPAD].reshape(NPAD, 1)
    p1 = deg2[NPAD:].reshape(NPAD, 1)
    hs2 = _tc2(h, stats, p0, p1, g2, be2)
    agg2 = _sc_agg(hs2, src2d, dst2d)
    return _tc3(agg2, p0, p1)


# speedup vs baseline: 21.0897x; 1.3037x over previous
import jax
import jax.numpy as jnp
from jax import lax
from jax.experimental import pallas as pl
from jax.experimental.pallas import tpu as pltpu
from jax.experimental.pallas import tpu_sc as plsc

N = 10000
E = 160000
C = 256
HALF = C // 2
EPS = 1e-5

NC = 2
NS = 16

ROW_BLK = 1000
GRID = N // ROW_BLK

EK = 125
EROWS = E // EK
DEG_RPW = EROWS // (NC * NS)
AGG_RPT = EROWS // NS
AGG_BATCH = 16

NPAD = 10240
STRIPE = NPAD // NS
SEED_CHUNK = 32


def _sc_mesh():
    return plsc.VectorSubcoreMesh(
        core_axis_name="c", subcore_axis_name="s", num_cores=NC, num_subcores=NS
    )


def _tc1_body(x_ref, wt_ref, b_ref, h_ref, stats_ref):
    i = pl.program_id(0)
    h = jnp.dot(x_ref[...], wt_ref[...], preferred_element_type=jnp.float32)
    h = h + b_ref[...]
    h_ref[...] = h
    s = jnp.sum(h, axis=0, keepdims=True)
    ss = jnp.sum(h * h, axis=0, keepdims=True)
    blk = jnp.concatenate([s, ss, jnp.zeros((6, C), jnp.float32)], axis=0)
    prev = jnp.where(i == 0, jnp.zeros_like(blk), stats_ref[...])
    stats_ref[...] = prev + blk


def _tc1(x, wt, b2):
    return pl.pallas_call(
        _tc1_body,
        grid=(GRID,),
        in_specs=[
            pl.BlockSpec((ROW_BLK, C), lambda i: (i, 0)),
            pl.BlockSpec((C, C), lambda i: (0, 0)),
            pl.BlockSpec((1, C), lambda i: (0, 0)),
        ],
        out_specs=[
            pl.BlockSpec((ROW_BLK, C), lambda i: (i, 0)),
            pl.BlockSpec((8, C), lambda i: (0, 0)),
        ],
        out_shape=[
            jax.ShapeDtypeStruct((N, C), jnp.float32),
            jax.ShapeDtypeStruct((8, C), jnp.float32),
        ],
    )(x, wt, b2)


def _deg_body(dst_hbm, deg_hbm, idx_v, ones_v, stage_v, deg_sh):
    c = lax.axis_index("c")
    s = lax.axis_index("s")
    w = c * NS + s

    one16 = jnp.ones((16,), jnp.float32)
    for k in range(8):
        ones_v[pl.ds(k * 16, 16)] = one16
    z16 = jnp.zeros((16,), jnp.float32)
    for k in range(STRIPE // 16):
        stage_v[pl.ds(k * 16, 16)] = z16

    off = pl.multiple_of(s * STRIPE, 8)
    pltpu.sync_copy(stage_v, deg_sh.at[pl.ds(off, STRIPE)])
    plsc.subcore_barrier()

    woff = pl.multiple_of(w * DEG_RPW, 8)
    pltpu.sync_copy(dst_hbm.at[pl.ds(woff, DEG_RPW)], idx_v)

    def body(j, carry):
        pltpu.sync_copy(ones_v.at[pl.ds(0, EK)],
                        deg_sh.at[idx_v.at[j]], add=True)
        return carry

    lax.fori_loop(0, DEG_RPW, body, 0)
    plsc.subcore_barrier()

    pltpu.sync_copy(deg_sh.at[pl.ds(off, STRIPE)], stage_v)
    goff = pl.multiple_of(c * NPAD + s * STRIPE, 8)
    pltpu.sync_copy(stage_v, deg_hbm.at[pl.ds(goff, STRIPE)])


def _sc_degree(dst_deg):
    return pl.kernel(
        _deg_body,
        out_type=jax.ShapeDtypeStruct((NC * NPAD,), jnp.float32),
        mesh=_sc_mesh(),
        scratch_types=[
            pltpu.VMEM((DEG_RPW, EK), jnp.int32),
            pltpu.VMEM((128,), jnp.float32),
            pltpu.VMEM((STRIPE,), jnp.float32),
            pltpu.VMEM_SHARED((NPAD,), jnp.float32),
        ],
    )(dst_deg)


def _tc2_body(h_ref, stats_ref, p0_ref, p1_ref, g_ref, be_ref, hs2_ref):
    inv_n = 1.0 / N
    mean = stats_ref[0:1, :] * inv_n
    var = stats_ref[1:2, :] * inv_n - mean * mean
    a = g_ref[...] * lax.rsqrt(var + EPS)
    cc = be_ref[...] - a * mean
    dinv = lax.rsqrt(p0_ref[...] + p1_ref[...] + 1.0)
    hs = (h_ref[...] * a + cc) * dinv
    hs2_ref[...] = jnp.stack([hs[:, :HALF], hs[:, HALF:]], axis=0)


def _tc2(h, stats, p0, p1, g2, be2):
    return pl.pallas_call(
        _tc2_body,
        grid=(GRID,),
        in_specs=[
            pl.BlockSpec((ROW_BLK, C), lambda i: (i, 0)),
            pl.BlockSpec((8, C), lambda i: (0, 0)),
            pl.BlockSpec((ROW_BLK, 1), lambda i: (i, 0)),
            pl.BlockSpec((ROW_BLK, 1), lambda i: (i, 0)),
            pl.BlockSpec((1, C), lambda i: (0, 0)),
            pl.BlockSpec((1, C), lambda i: (0, 0)),
        ],
        out_specs=pl.BlockSpec((NC, ROW_BLK, HALF), lambda i: (0, i, 0)),
        out_shape=jax.ShapeDtypeStruct((NC, NPAD, HALF), jnp.float32),
    )(h, stats, p0, p1, g2, be2)


def _agg_body(hs2_hbm, src_hbm, dst_hbm, agg2_hbm,
              idx_s0, idx_s1, idx_d0, idx_d1, rows0, rows1, stage, agg_sh,
              gsem0, gsem1, ssem0, ssem1):
    c = lax.axis_index("c")
    s = lax.axis_index("s")
    off = pl.multiple_of(s * STRIPE, 8)
    hs_ref = hs2_hbm.at[c]
    out_ref = agg2_hbm.at[c]
    idx_s = (idx_s0, idx_s1)
    idx_d = (idx_d0, idx_d1)
    rows = (rows0, rows1)
    gsem = (gsem0, gsem1)
    ssem = (ssem0, ssem1)

    for k in range(STRIPE // SEED_CHUNK):
        koff = pl.multiple_of(off + k * SEED_CHUNK, 8)
        pltpu.sync_copy(hs_ref.at[pl.ds(koff, SEED_CHUNK)], stage)
        pltpu.sync_copy(stage, agg_sh.at[pl.ds(koff, SEED_CHUNK)])
    plsc.subcore_barrier()

    gd = [None, None]
    sd = [None, None]
    for j in range(AGG_RPT + 1):
        slot = j % 2
        if j < AGG_RPT:
            b, r = divmod(j, AGG_BATCH)
            if r == 0:
                boff = pl.multiple_of(s * AGG_RPT + b * AGG_BATCH, 8)
                pltpu.sync_copy(src_hbm.at[pl.ds(boff, AGG_BATCH)],
                                idx_s[b % 2])
                pltpu.sync_copy(dst_hbm.at[pl.ds(boff, AGG_BATCH)],
                                idx_d[b % 2])
            if sd[slot] is not None:
                sd[slot].wait()
            gd[slot] = pltpu.async_copy(
                hs_ref.at[idx_s[b % 2].at[r]], rows[slot], gsem[slot])
        if j > 0:
            pslot = (j - 1) % 2
            pb, pr = divmod(j - 1, AGG_BATCH)
            gd[pslot].wait()
            sd[pslot] = pltpu.async_copy(
                rows[pslot], agg_sh.at[idx_d[pb % 2].at[pr]], ssem[pslot],
                add=True)
    sd[(AGG_RPT - 1) % 2].wait()
    sd[AGG_RPT % 2].wait()
    plsc.subcore_barrier()

    for k in range(STRIPE // SEED_CHUNK):
        koff = pl.multiple_of(off + k * SEED_CHUNK, 8)
        pltpu.sync_copy(agg_sh.at[pl.ds(koff, SEED_CHUNK)], stage)
        pltpu.sync_copy(stage, out_ref.at[pl.ds(koff, SEED_CHUNK)])


def _sc_agg(hs2, src_agg, dst_agg):
    return pl.kernel(
        _agg_body,
        out_type=jax.ShapeDtypeStruct((NC, NPAD, HALF), jnp.float32),
        mesh=_sc_mesh(),
        scratch_types=[
            pltpu.VMEM((AGG_BATCH, EK), jnp.int32),
            pltpu.VMEM((AGG_BATCH, EK), jnp.int32),
            pltpu.VMEM((AGG_BATCH, EK), jnp.int32),
            pltpu.VMEM((AGG_BATCH, EK), jnp.int32),
            pltpu.VMEM((EK, HALF), jnp.float32),
            pltpu.VMEM((EK, HALF), jnp.float32),
            pltpu.VMEM((SEED_CHUNK, HALF), jnp.float32),
            pltpu.VMEM_SHARED((NPAD, HALF), jnp.float32),
            pltpu.SemaphoreType.DMA,
            pltpu.SemaphoreType.DMA,
            pltpu.SemaphoreType.DMA,
            pltpu.SemaphoreType.DMA,
        ],
    )(hs2, src_agg, dst_agg)


def _tc3_body(a2_ref, p0_ref, p1_ref, out_ref):
    dinv = lax.rsqrt(p0_ref[...] + p1_ref[...] + 1.0)
    a2 = a2_ref[...]
    o = jnp.concatenate([a2[0], a2[1]], axis=1) * dinv
    out_ref[...] = jnp.maximum(o, 0.0)


def _tc3(agg2, p0, p1):
    return pl.pallas_call(
        _tc3_body,
        grid=(GRID,),
        in_specs=[
            pl.BlockSpec((NC, ROW_BLK, HALF), lambda i: (0, i, 0)),
            pl.BlockSpec((ROW_BLK, 1), lambda i: (i, 0)),
            pl.BlockSpec((ROW_BLK, 1), lambda i: (i, 0)),
        ],
        out_specs=pl.BlockSpec((ROW_BLK, C), lambda i: (i, 0)),
        out_shape=jax.ShapeDtypeStruct((N, C), jnp.float32),
    )(agg2, p0, p1)


def kernel(X, edge_index, W, b, gamma, beta):
    src = edge_index[0]
    dst = edge_index[1]
    src2d = src.reshape(EROWS, EK)
    dst2d = dst.reshape(EROWS, EK)
    wt = W.T
    b2 = b.reshape(1, C)
    g2 = gamma.reshape(1, C)
    be2 = beta.reshape(1, C)

    h, stats = _tc1(X, wt, b2)
    deg2 = _sc_degree(dst2d)
    p0 = deg2[:NPAD].reshape(NPAD, 1)
    p1 = deg2[NPAD:].reshape(NPAD, 1)
    hs2 = _tc2(h, stats, p0, p1, g2, be2)
    agg2 = _sc_agg(hs2, src2d, dst2d)
    return _tc3(agg2, p0, p1)


# speedup vs baseline: 21.1668x; 1.0037x over previous
import jax
import jax.numpy as jnp
from jax import lax
from jax.experimental import pallas as pl
from jax.experimental.pallas import tpu as pltpu
from jax.experimental.pallas import tpu_sc as plsc

N = 10000
E = 160000
C = 256
HALF = C // 2
EPS = 1e-5

NC = 2
NS = 16

ROW_BLK = 1000
GRID = N // ROW_BLK

EK = 125
EROWS = E // EK
DEG_RPW = EROWS // (NC * NS)
AGG_RPT = EROWS // NS
AGG_BATCH = 16

NPAD = 10240
STRIPE = NPAD // NS
SEED_CHUNK = 32


def _sc_mesh():
    return plsc.VectorSubcoreMesh(
        core_axis_name="c", subcore_axis_name="s", num_cores=NC, num_subcores=NS
    )


def _tca_body(x_ref, wt_ref, b_ref, p0_ref, p1_ref, g_ref, be_ref,
              hs2_ref, h_scr, stats_scr):
    i = pl.program_id(0)

    @pl.when(i < GRID)
    def _phase1():
        h = jnp.dot(x_ref[...], wt_ref[...],
                    preferred_element_type=jnp.float32)
        h = h + b_ref[...]
        h_scr[pl.ds(i * ROW_BLK, ROW_BLK), :] = h
        s = jnp.sum(h, axis=0, keepdims=True)
        ss = jnp.sum(h * h, axis=0, keepdims=True)
        blk = jnp.concatenate(
            [s, ss, jnp.zeros((6, C), jnp.float32)], axis=0)
        prev = jnp.where(i == 0, jnp.zeros_like(blk), stats_scr[...])
        stats_scr[...] = prev + blk

    @pl.when(i >= GRID)
    def _phase2():
        r = i - GRID
        inv_n = 1.0 / N
        mean = stats_scr[0:1, :] * inv_n
        var = stats_scr[1:2, :] * inv_n - mean * mean
        a = g_ref[...] * lax.rsqrt(var + EPS)
        cc = be_ref[...] - a * mean
        dinv = lax.rsqrt(p0_ref[...] + p1_ref[...] + 1.0)
        h = h_scr[pl.ds(r * ROW_BLK, ROW_BLK), :]
        hs = (h * a + cc) * dinv
        hs2_ref[...] = jnp.stack([hs[:, :HALF], hs[:, HALF:]], axis=0)


def _tca(x, wt, b2, p0, p1, g2, be2):
    def x_map(i):
        return (lax.rem(i, GRID), 0)

    def row_map(i):
        return (jnp.where(i < GRID, 0, i - GRID), 0)

    def hs_map(i):
        return (0, jnp.where(i < GRID, 0, i - GRID), 0)

    return pl.pallas_call(
        _tca_body,
        grid=(2 * GRID,),
        in_specs=[
            pl.BlockSpec((ROW_BLK, C), x_map),
            pl.BlockSpec((C, C), lambda i: (0, 0)),
            pl.BlockSpec((1, C), lambda i: (0, 0)),
            pl.BlockSpec((ROW_BLK, 1), row_map),
            pl.BlockSpec((ROW_BLK, 1), row_map),
            pl.BlockSpec((1, C), lambda i: (0, 0)),
            pl.BlockSpec((1, C), lambda i: (0, 0)),
        ],
        out_specs=pl.BlockSpec((NC, ROW_BLK, HALF), hs_map),
        out_shape=jax.ShapeDtypeStruct((NC, NPAD, HALF), jnp.float32),
        scratch_shapes=[
            pltpu.VMEM((N, C), jnp.float32),
            pltpu.VMEM((8, C), jnp.float32),
        ],
    )(x, wt, b2, p0, p1, g2, be2)


def _deg_body(dst_hbm, deg_hbm, idx_v, ones_v, stage_v, deg_sh):
    c = lax.axis_index("c")
    s = lax.axis_index("s")
    w = c * NS + s

    one16 = jnp.ones((16,), jnp.float32)
    for k in range(8):
        ones_v[pl.ds(k * 16, 16)] = one16
    z16 = jnp.zeros((16,), jnp.float32)
    for k in range(STRIPE // 16):
        stage_v[pl.ds(k * 16, 16)] = z16

    off = pl.multiple_of(s * STRIPE, 8)
    pltpu.sync_copy(stage_v, deg_sh.at[pl.ds(off, STRIPE)])
    plsc.subcore_barrier()

    woff = pl.multiple_of(w * DEG_RPW, 8)
    pltpu.sync_copy(dst_hbm.at[pl.ds(woff, DEG_RPW)], idx_v)

    def body(j, carry):
        pltpu.sync_copy(ones_v.at[pl.ds(0, EK)],
                        deg_sh.at[idx_v.at[j]], add=True)
        return carry

    lax.fori_loop(0, DEG_RPW, body, 0)
    plsc.subcore_barrier()

    pltpu.sync_copy(deg_sh.at[pl.ds(off, STRIPE)], stage_v)
    goff = pl.multiple_of(c * NPAD + s * STRIPE, 8)
    pltpu.sync_copy(stage_v, deg_hbm.at[pl.ds(goff, STRIPE)])


def _sc_degree(dst_deg):
    return pl.kernel(
        _deg_body,
        out_type=jax.ShapeDtypeStruct((NC * NPAD,), jnp.float32),
        mesh=_sc_mesh(),
        scratch_types=[
            pltpu.VMEM((DEG_RPW, EK), jnp.int32),
            pltpu.VMEM((128,), jnp.float32),
            pltpu.VMEM((STRIPE,), jnp.float32),
            pltpu.VMEM_SHARED((NPAD,), jnp.float32),
        ],
    )(dst_deg)


def _agg_body(hs2_hbm, src_hbm, dst_hbm, agg2_hbm,
              idx_s0, idx_s1, idx_d0, idx_d1, rows0, rows1, stage, agg_sh,
              gsem0, gsem1, ssem0, ssem1):
    c = lax.axis_index("c")
    s = lax.axis_index("s")
    off = pl.multiple_of(s * STRIPE, 8)
    hs_ref = hs2_hbm.at[c]
    out_ref = agg2_hbm.at[c]
    idx_s = (idx_s0, idx_s1)
    idx_d = (idx_d0, idx_d1)
    rows = (rows0, rows1)
    gsem = (gsem0, gsem1)
    ssem = (ssem0, ssem1)

    for k in range(STRIPE // SEED_CHUNK):
        koff = pl.multiple_of(off + k * SEED_CHUNK, 8)
        pltpu.sync_copy(hs_ref.at[pl.ds(koff, SEED_CHUNK)], stage)
        pltpu.sync_copy(stage, agg_sh.at[pl.ds(koff, SEED_CHUNK)])
    plsc.subcore_barrier()

    gd = [None, None]
    sd = [None, None]
    for j in range(AGG_RPT + 1):
        slot = j % 2
        if j < AGG_RPT:
            b, r = divmod(j, AGG_BATCH)
            if r == 0:
                boff = pl.multiple_of(s * AGG_RPT + b * AGG_BATCH, 8)
                pltpu.sync_copy(src_hbm.at[pl.ds(boff, AGG_BATCH)],
                                idx_s[b % 2])
                pltpu.sync_copy(dst_hbm.at[pl.ds(boff, AGG_BATCH)],
                                idx_d[b % 2])
            if sd[slot] is not None:
                sd[slot].wait()
            gd[slot] = pltpu.async_copy(
                hs_ref.at[idx_s[b % 2].at[r]], rows[slot], gsem[slot])
        if j > 0:
            pslot = (j - 1) % 2
            pb, pr = divmod(j - 1, AGG_BATCH)
            gd[pslot].wait()
            sd[pslot] = pltpu.async_copy(
                rows[pslot], agg_sh.at[idx_d[pb % 2].at[pr]], ssem[pslot],
                add=True)
    sd[(AGG_RPT - 1) % 2].wait()
    sd[AGG_RPT % 2].wait()
    plsc.subcore_barrier()

    for k in range(STRIPE // SEED_CHUNK):
        koff = pl.multiple_of(off + k * SEED_CHUNK, 8)
        pltpu.sync_copy(agg_sh.at[pl.ds(koff, SEED_CHUNK)], stage)
        pltpu.sync_copy(stage, out_ref.at[pl.ds(koff, SEED_CHUNK)])


def _sc_agg(hs2, src_agg, dst_agg):
    return pl.kernel(
        _agg_body,
        out_type=jax.ShapeDtypeStruct((NC, NPAD, HALF), jnp.float32),
        mesh=_sc_mesh(),
        scratch_types=[
            pltpu.VMEM((AGG_BATCH, EK), jnp.int32),
            pltpu.VMEM((AGG_BATCH, EK), jnp.int32),
            pltpu.VMEM((AGG_BATCH, EK), jnp.int32),
            pltpu.VMEM((AGG_BATCH, EK), jnp.int32),
            pltpu.VMEM((EK, HALF), jnp.float32),
            pltpu.VMEM((EK, HALF), jnp.float32),
            pltpu.VMEM((SEED_CHUNK, HALF), jnp.float32),
            pltpu.VMEM_SHARED((NPAD, HALF), jnp.float32),
            pltpu.SemaphoreType.DMA,
            pltpu.SemaphoreType.DMA,
            pltpu.SemaphoreType.DMA,
            pltpu.SemaphoreType.DMA,
        ],
    )(hs2, src_agg, dst_agg)


def _tc3_body(a2_ref, p0_ref, p1_ref, out_ref):
    dinv = lax.rsqrt(p0_ref[...] + p1_ref[...] + 1.0)
    a2 = a2_ref[...]
    o = jnp.concatenate([a2[0], a2[1]], axis=1) * dinv
    out_ref[...] = jnp.maximum(o, 0.0)


def _tc3(agg2, p0, p1):
    return pl.pallas_call(
        _tc3_body,
        grid=(GRID,),
        in_specs=[
            pl.BlockSpec((NC, ROW_BLK, HALF), lambda i: (0, i, 0)),
            pl.BlockSpec((ROW_BLK, 1), lambda i: (i, 0)),
            pl.BlockSpec((ROW_BLK, 1), lambda i: (i, 0)),
        ],
        out_specs=pl.BlockSpec((ROW_BLK, C), lambda i: (i, 0)),
        out_shape=jax.ShapeDtypeStruct((N, C), jnp.float32),
    )(agg2, p0, p1)


def kernel(X, edge_index, W, b, gamma, beta):
    src = edge_index[0]
    dst = edge_index[1]
    src2d = src.reshape(EROWS, EK)
    dst2d = dst.reshape(EROWS, EK)
    wt = W.T
    b2 = b.reshape(1, C)
    g2 = gamma.reshape(1, C)
    be2 = beta.reshape(1, C)

    deg2 = _sc_degree(dst2d)
    p0 = deg2[:NPAD].reshape(NPAD, 1)
    p1 = deg2[NPAD:].reshape(NPAD, 1)
    hs2 = _tca(X, wt, b2, p0, p1, g2, be2)
    agg2 = _sc_agg(hs2, src2d, dst2d)
    return _tc3(agg2, p0, p1)


# speedup vs baseline: 21.6967x; 1.0250x over previous
import jax
import jax.numpy as jnp
from jax import lax
from jax.experimental import pallas as pl
from jax.experimental.pallas import tpu as pltpu
from jax.experimental.pallas import tpu_sc as plsc

N = 10000
E = 160000
C = 256
HALF = C // 2
EPS = 1e-5

NC = 2
NS = 16

ROW_BLK = 1000
GRID = N // ROW_BLK

EK = 125
EROWS = E // EK
DEG_RPW = EROWS // NS
AGG_RPT = EROWS // NS
AGG_BATCH = 16

NPAD = 10240
STRIPE = NPAD // NS
SEED_CHUNK = 32


def _sc_mesh():
    return plsc.VectorSubcoreMesh(
        core_axis_name="c", subcore_axis_name="s", num_cores=NC, num_subcores=NS
    )


def _tca_body(x_ref, wt_ref, b_ref, p_ref, g_ref, be_ref,
              hs2_ref, h_scr, stats_scr):
    i = pl.program_id(0)

    @pl.when(i < GRID)
    def _phase1():
        h = jnp.dot(x_ref[...], wt_ref[...],
                    preferred_element_type=jnp.float32)
        h = h + b_ref[...]
        h_scr[pl.ds(i * ROW_BLK, ROW_BLK), :] = h
        s = jnp.sum(h, axis=0, keepdims=True)
        ss = jnp.sum(h * h, axis=0, keepdims=True)
        blk = jnp.concatenate(
            [s, ss, jnp.zeros((6, C), jnp.float32)], axis=0)
        prev = jnp.where(i == 0, jnp.zeros_like(blk), stats_scr[...])
        stats_scr[...] = prev + blk

    @pl.when(i >= GRID)
    def _phase2():
        r = i - GRID
        inv_n = 1.0 / N
        mean = stats_scr[0:1, :] * inv_n
        var = stats_scr[1:2, :] * inv_n - mean * mean
        a = g_ref[...] * lax.rsqrt(var + EPS)
        cc = be_ref[...] - a * mean
        dinv = lax.rsqrt(p_ref[...] + 1.0)
        h = h_scr[pl.ds(r * ROW_BLK, ROW_BLK), :]
        hs = (h * a + cc) * dinv
        hs2_ref[...] = jnp.stack([hs[:, :HALF], hs[:, HALF:]], axis=0)


def _tca(x, wt, b2, p, g2, be2):
    def x_map(i):
        return (lax.rem(i, GRID), 0)

    def row_map(i):
        return (jnp.where(i < GRID, 0, i - GRID), 0)

    def hs_map(i):
        return (0, jnp.where(i < GRID, 0, i - GRID), 0)

    return pl.pallas_call(
        _tca_body,
        grid=(2 * GRID,),
        in_specs=[
            pl.BlockSpec((ROW_BLK, C), x_map),
            pl.BlockSpec((C, C), lambda i: (0, 0)),
            pl.BlockSpec((1, C), lambda i: (0, 0)),
            pl.BlockSpec((ROW_BLK, 1), row_map),
            pl.BlockSpec((1, C), lambda i: (0, 0)),
            pl.BlockSpec((1, C), lambda i: (0, 0)),
        ],
        out_specs=pl.BlockSpec((NC, ROW_BLK, HALF), hs_map),
        out_shape=jax.ShapeDtypeStruct((NC, NPAD, HALF), jnp.float32),
        scratch_shapes=[
            pltpu.VMEM((N, C), jnp.float32),
            pltpu.VMEM((8, C), jnp.float32),
        ],
    )(x, wt, b2, p, g2, be2)


def _deg_body(edge_hbm, deg_hbm, idx_v, ones_v, stage_v, deg_sh):
    c = lax.axis_index("c")
    s = lax.axis_index("s")
    dst_hbm = edge_hbm.at[1]

    one16 = jnp.ones((16,), jnp.float32)
    for k in range(8):
        ones_v[pl.ds(k * 16, 16)] = one16
    z16 = jnp.zeros((16,), jnp.float32)
    for k in range(STRIPE // 16):
        stage_v[pl.ds(k * 16, 16)] = z16

    off = pl.multiple_of(s * STRIPE, 8)
    pltpu.sync_copy(stage_v, deg_sh.at[pl.ds(off, STRIPE)])
    plsc.subcore_barrier()

    woff = pl.multiple_of(s * DEG_RPW, 8)
    pltpu.sync_copy(dst_hbm.at[pl.ds(woff, DEG_RPW)], idx_v)

    def body(j, carry):
        pltpu.sync_copy(ones_v.at[pl.ds(0, EK)],
                        deg_sh.at[idx_v.at[j]], add=True)
        return carry

    lax.fori_loop(0, DEG_RPW, body, 0)
    plsc.subcore_barrier()

    @pl.when(c == 0)
    def _():
        pltpu.sync_copy(deg_sh.at[pl.ds(off, STRIPE)], stage_v)
        pltpu.sync_copy(stage_v, deg_hbm.at[pl.ds(off, STRIPE)])


def _sc_degree(edge2):
    return pl.kernel(
        _deg_body,
        out_type=jax.ShapeDtypeStruct((NPAD,), jnp.float32),
        mesh=_sc_mesh(),
        scratch_types=[
            pltpu.VMEM((DEG_RPW, EK), jnp.int32),
            pltpu.VMEM((128,), jnp.float32),
            pltpu.VMEM((STRIPE,), jnp.float32),
            pltpu.VMEM_SHARED((NPAD,), jnp.float32),
        ],
    )(edge2)


def _agg_body(hs2_hbm, edge_hbm, agg2_hbm,
              idx_s0, idx_s1, idx_d0, idx_d1, rows0, rows1, stage, agg_sh,
              gsem0, gsem1, ssem0, ssem1):
    c = lax.axis_index("c")
    s = lax.axis_index("s")
    off = pl.multiple_of(s * STRIPE, 8)
    hs_ref = hs2_hbm.at[c]
    out_ref = agg2_hbm.at[c]
    src_hbm = edge_hbm.at[0]
    dst_hbm = edge_hbm.at[1]
    idx_s = (idx_s0, idx_s1)
    idx_d = (idx_d0, idx_d1)
    rows = (rows0, rows1)
    gsem = (gsem0, gsem1)
    ssem = (ssem0, ssem1)

    for k in range(STRIPE // SEED_CHUNK):
        koff = pl.multiple_of(off + k * SEED_CHUNK, 8)
        pltpu.sync_copy(hs_ref.at[pl.ds(koff, SEED_CHUNK)], stage)
        pltpu.sync_copy(stage, agg_sh.at[pl.ds(koff, SEED_CHUNK)])
    plsc.subcore_barrier()

    gd = [None, None]
    sd = [None, None]
    for j in range(AGG_RPT + 1):
        slot = j % 2
        if j < AGG_RPT:
            b, r = divmod(j, AGG_BATCH)
            if r == 0:
                boff = pl.multiple_of(s * AGG_RPT + b * AGG_BATCH, 8)
                pltpu.sync_copy(src_hbm.at[pl.ds(boff, AGG_BATCH)],
                                idx_s[b % 2])
                pltpu.sync_copy(dst_hbm.at[pl.ds(boff, AGG_BATCH)],
                                idx_d[b % 2])
            if sd[slot] is not None:
                sd[slot].wait()
            gd[slot] = pltpu.async_copy(
                hs_ref.at[idx_s[b % 2].at[r]], rows[slot], gsem[slot])
        if j > 0:
            pslot = (j - 1) % 2
            pb, pr = divmod(j - 1, AGG_BATCH)
            gd[pslot].wait()
            sd[pslot] = pltpu.async_copy(
                rows[pslot], agg_sh.at[idx_d[pb % 2].at[pr]], ssem[pslot],
                add=True)
    sd[(AGG_RPT - 1) % 2].wait()
    sd[AGG_RPT % 2].wait()
    plsc.subcore_barrier()

    for k in range(STRIPE // SEED_CHUNK):
        koff = pl.multiple_of(off + k * SEED_CHUNK, 8)
        pltpu.sync_copy(agg_sh.at[pl.ds(koff, SEED_CHUNK)], stage)
        pltpu.sync_copy(stage, out_ref.at[pl.ds(koff, SEED_CHUNK)])


def _sc_agg(hs2, edge2):
    return pl.kernel(
        _agg_body,
        out_type=jax.ShapeDtypeStruct((NC, NPAD, HALF), jnp.float32),
        mesh=_sc_mesh(),
        scratch_types=[
            pltpu.VMEM((AGG_BATCH, EK), jnp.int32),
            pltpu.VMEM((AGG_BATCH, EK), jnp.int32),
            pltpu.VMEM((AGG_BATCH, EK), jnp.int32),
            pltpu.VMEM((AGG_BATCH, EK), jnp.int32),
            pltpu.VMEM((EK, HALF), jnp.float32),
            pltpu.VMEM((EK, HALF), jnp.float32),
            pltpu.VMEM((SEED_CHUNK, HALF), jnp.float32),
            pltpu.VMEM_SHARED((NPAD, HALF), jnp.float32),
            pltpu.SemaphoreType.DMA,
            pltpu.SemaphoreType.DMA,
            pltpu.SemaphoreType.DMA,
            pltpu.SemaphoreType.DMA,
        ],
    )(hs2, edge2)


def _tc3_body(a2_ref, p_ref, out_ref):
    dinv = lax.rsqrt(p_ref[...] + 1.0)
    a2 = a2_ref[...]
    o = jnp.concatenate([a2[0], a2[1]], axis=1) * dinv
    out_ref[...] = jnp.maximum(o, 0.0)


def _tc3(agg2, p):
    return pl.pallas_call(
        _tc3_body,
        grid=(GRID,),
        in_specs=[
            pl.BlockSpec((NC, ROW_BLK, HALF), lambda i: (0, i, 0)),
            pl.BlockSpec((ROW_BLK, 1), lambda i: (i, 0)),
        ],
        out_specs=pl.BlockSpec((ROW_BLK, C), lambda i: (i, 0)),
        out_shape=jax.ShapeDtypeStruct((N, C), jnp.float32),
    )(agg2, p)


def kernel(X, edge_index, W, b, gamma, beta):
    edge2 = edge_index.reshape(2, EROWS, EK)
    wt = W.T
    b2 = b.reshape(1, C)
    g2 = gamma.reshape(1, C)
    be2 = beta.reshape(1, C)

    deg = _sc_degree(edge2)
    p = deg.reshape(NPAD, 1)
    hs2 = _tca(X, wt, b2, p, g2, be2)
    agg2 = _sc_agg(hs2, edge2)
    return _tc3(agg2, p)


# speedup vs baseline: 21.8961x; 1.0092x over previous
import jax
import jax.numpy as jnp
from jax import lax
from jax.experimental import pallas as pl
from jax.experimental.pallas import tpu as pltpu
from jax.experimental.pallas import tpu_sc as plsc

N = 10000
E = 160000
C = 256
HALF = C // 2
EPS = 1e-5

NC = 2
NS = 16

ROW_BLK = 1000
GRID = N // ROW_BLK

EK = 125
EROWS = E // EK
DEG_RPW = EROWS // NS
AGG_RPT = EROWS // NS
AGG_BATCH = 16

NPAD = 10240
STRIPE = NPAD // NS
SEED_CHUNK = 32


def _sc_mesh():
    return plsc.VectorSubcoreMesh(
        core_axis_name="c", subcore_axis_name="s", num_cores=NC, num_subcores=NS
    )


def _tca_body(x_ref, wt_ref, b_ref, p_ref, g_ref, be_ref,
              hs2_ref, h_scr, stats_scr):
    i = pl.program_id(0)

    @pl.when(i < GRID)
    def _phase1():
        h = jnp.dot(x_ref[...], wt_ref[...],
                    preferred_element_type=jnp.float32)
        h = h + b_ref[...]
        h_scr[pl.ds(i * ROW_BLK, ROW_BLK), :] = h
        s = jnp.sum(h, axis=0, keepdims=True)
        ss = jnp.sum(h * h, axis=0, keepdims=True)
        blk = jnp.concatenate(
            [s, ss, jnp.zeros((6, C), jnp.float32)], axis=0)
        prev = jnp.where(i == 0, jnp.zeros_like(blk), stats_scr[...])
        stats_scr[...] = prev + blk

    @pl.when(i >= GRID)
    def _phase2():
        r = i - GRID
        inv_n = 1.0 / N
        mean = stats_scr[0:1, :] * inv_n
        var = stats_scr[1:2, :] * inv_n - mean * mean
        a = g_ref[...] * lax.rsqrt(var + EPS)
        cc = be_ref[...] - a * mean
        dinv = lax.rsqrt(p_ref[...] + 1.0)
        h = h_scr[pl.ds(r * ROW_BLK, ROW_BLK), :]
        hs = (h * a + cc) * dinv
        hs2_ref[...] = jnp.stack([hs[:, :HALF], hs[:, HALF:]], axis=0)


def _tca(x, wt, b2, p, g2, be2):
    def x_map(i):
        return (lax.rem(i, GRID), 0)

    def row_map(i):
        return (jnp.where(i < GRID, 0, i - GRID), 0)

    def hs_map(i):
        return (0, jnp.where(i < GRID, 0, i - GRID), 0)

    return pl.pallas_call(
        _tca_body,
        grid=(2 * GRID,),
        in_specs=[
            pl.BlockSpec((ROW_BLK, C), x_map),
            pl.BlockSpec((C, C), lambda i: (0, 0)),
            pl.BlockSpec((1, C), lambda i: (0, 0)),
            pl.BlockSpec((ROW_BLK, 1), row_map),
            pl.BlockSpec((1, C), lambda i: (0, 0)),
            pl.BlockSpec((1, C), lambda i: (0, 0)),
        ],
        out_specs=pl.BlockSpec((NC, ROW_BLK, HALF), hs_map),
        out_shape=jax.ShapeDtypeStruct((NC, NPAD, HALF), jnp.float32),
        scratch_shapes=[
            pltpu.VMEM((N, C), jnp.float32),
            pltpu.VMEM((8, C), jnp.float32),
        ],
    )(x, wt, b2, p, g2, be2)


def _deg_body(edge_hbm, deg_hbm, idx_v, ones_v, stage_v, deg_sh):
    c = lax.axis_index("c")
    s = lax.axis_index("s")
    dst_hbm = edge_hbm.at[1]

    one16 = jnp.ones((16,), jnp.float32)
    for k in range(8):
        ones_v[pl.ds(k * 16, 16)] = one16
    z16 = jnp.zeros((16,), jnp.float32)
    for k in range(STRIPE // 16):
        stage_v[pl.ds(k * 16, 16)] = z16

    off = pl.multiple_of(s * STRIPE, 8)
    pltpu.sync_copy(stage_v, deg_sh.at[pl.ds(off, STRIPE)])
    plsc.subcore_barrier()

    woff = pl.multiple_of(s * DEG_RPW, 8)
    pltpu.sync_copy(dst_hbm.at[pl.ds(woff, DEG_RPW)], idx_v)

    def body(j, carry):
        pltpu.sync_copy(ones_v.at[pl.ds(0, EK)],
                        deg_sh.at[idx_v.at[j]], add=True)
        return carry

    lax.fori_loop(0, DEG_RPW, body, 0)
    plsc.subcore_barrier()

    @pl.when(c == 0)
    def _():
        pltpu.sync_copy(deg_sh.at[pl.ds(off, STRIPE)], stage_v)
        pltpu.sync_copy(stage_v, deg_hbm.at[pl.ds(off, STRIPE)])


def _sc_degree(edge2):
    return pl.kernel(
        _deg_body,
        out_type=jax.ShapeDtypeStruct((NPAD,), jnp.float32),
        mesh=_sc_mesh(),
        scratch_types=[
            pltpu.VMEM((DEG_RPW, EK), jnp.int32),
            pltpu.VMEM((128,), jnp.float32),
            pltpu.VMEM((STRIPE,), jnp.float32),
            pltpu.VMEM_SHARED((NPAD,), jnp.float32),
        ],
    )(edge2)


def _agg_body(hs2_hbm, edge_hbm, agg2_hbm,
              idx_s0, idx_s1, idx_d0, idx_d1, rows0, rows1, stage, agg_sh,
              gsem0, gsem1, ssem0, ssem1):
    c = lax.axis_index("c")
    s = lax.axis_index("s")
    off = pl.multiple_of(s * STRIPE, 8)
    hs_ref = hs2_hbm.at[c]
    out_ref = agg2_hbm.at[c]
    src_hbm = edge_hbm.at[0]
    dst_hbm = edge_hbm.at[1]
    idx_s = (idx_s0, idx_s1)
    idx_d = (idx_d0, idx_d1)
    rows = (rows0, rows1)
    gsem = (gsem0, gsem1)
    ssem = (ssem0, ssem1)

    boff0 = pl.multiple_of(s * AGG_RPT, 8)
    pltpu.sync_copy(src_hbm.at[pl.ds(boff0, AGG_BATCH)], idx_s[0])
    pltpu.sync_copy(dst_hbm.at[pl.ds(boff0, AGG_BATCH)], idx_d[0])
    gd = [None, None]
    sd = [None, None]
    gd[0] = pltpu.async_copy(hs_ref.at[idx_s[0].at[0]], rows[0], gsem[0])
    gd[1] = pltpu.async_copy(hs_ref.at[idx_s[0].at[1]], rows[1], gsem[1])
    for k in range(STRIPE // SEED_CHUNK):
        koff = pl.multiple_of(off + k * SEED_CHUNK, 8)
        pltpu.sync_copy(hs_ref.at[pl.ds(koff, SEED_CHUNK)], stage)
        pltpu.sync_copy(stage, agg_sh.at[pl.ds(koff, SEED_CHUNK)])
    plsc.subcore_barrier()

    for j in range(AGG_RPT):
        slot = j % 2
        b, r = divmod(j, AGG_BATCH)
        gd[slot].wait()
        sd[slot] = pltpu.async_copy(
            rows[slot], agg_sh.at[idx_d[b % 2].at[r]], ssem[slot],
            add=True)
        nj = j + 2
        if nj < AGG_RPT:
            nb, nr = divmod(nj, AGG_BATCH)
            if nr == 0:
                boff = pl.multiple_of(s * AGG_RPT + nb * AGG_BATCH, 8)
                pltpu.sync_copy(src_hbm.at[pl.ds(boff, AGG_BATCH)],
                                idx_s[nb % 2])
                pltpu.sync_copy(dst_hbm.at[pl.ds(boff, AGG_BATCH)],
                                idx_d[nb % 2])
            sd[slot].wait()
            gd[slot] = pltpu.async_copy(
                hs_ref.at[idx_s[nb % 2].at[nr]], rows[slot], gsem[slot])
    sd[(AGG_RPT - 1) % 2].wait()
    sd[AGG_RPT % 2].wait()
    plsc.subcore_barrier()

    for k in range(STRIPE // SEED_CHUNK):
        koff = pl.multiple_of(off + k * SEED_CHUNK, 8)
        pltpu.sync_copy(agg_sh.at[pl.ds(koff, SEED_CHUNK)], stage)
        pltpu.sync_copy(stage, out_ref.at[pl.ds(koff, SEED_CHUNK)])


def _sc_agg(hs2, edge2):
    return pl.kernel(
        _agg_body,
        out_type=jax.ShapeDtypeStruct((NC, NPAD, HALF), jnp.float32),
        mesh=_sc_mesh(),
        scratch_types=[
            pltpu.VMEM((AGG_BATCH, EK), jnp.int32),
            pltpu.VMEM((AGG_BATCH, EK), jnp.int32),
            pltpu.VMEM((AGG_BATCH, EK), jnp.int32),
            pltpu.VMEM((AGG_BATCH, EK), jnp.int32),
            pltpu.VMEM((EK, HALF), jnp.float32),
            pltpu.VMEM((EK, HALF), jnp.float32),
            pltpu.VMEM((SEED_CHUNK, HALF), jnp.float32),
            pltpu.VMEM_SHARED((NPAD, HALF), jnp.float32),
            pltpu.SemaphoreType.DMA,
            pltpu.SemaphoreType.DMA,
            pltpu.SemaphoreType.DMA,
            pltpu.SemaphoreType.DMA,
        ],
    )(hs2, edge2)


def _tc3_body(a2_ref, p_ref, out_ref):
    dinv = lax.rsqrt(p_ref[...] + 1.0)
    a2 = a2_ref[...]
    o = jnp.concatenate([a2[0], a2[1]], axis=1) * dinv
    out_ref[...] = jnp.maximum(o, 0.0)


def _tc3(agg2, p):
    return pl.pallas_call(
        _tc3_body,
        grid=(GRID,),
        in_specs=[
            pl.BlockSpec((NC, ROW_BLK, HALF), lambda i: (0, i, 0)),
            pl.BlockSpec((ROW_BLK, 1), lambda i: (i, 0)),
        ],
        out_specs=pl.BlockSpec((ROW_BLK, C), lambda i: (i, 0)),
        out_shape=jax.ShapeDtypeStruct((N, C), jnp.float32),
    )(agg2, p)


def kernel(X, edge_index, W, b, gamma, beta):
    edge2 = edge_index.reshape(2, EROWS, EK)
    wt = W.T
    b2 = b.reshape(1, C)
    g2 = gamma.reshape(1, C)
    be2 = beta.reshape(1, C)

    deg = _sc_degree(edge2)
    p = deg.reshape(NPAD, 1)
    hs2 = _tca(X, wt, b2, p, g2, be2)
    agg2 = _sc_agg(hs2, edge2)
    return _tc3(agg2, p)


# speedup vs baseline: 22.4747x; 1.0264x over previous
import jax
import jax.numpy as jnp
from jax import lax
from jax.experimental import pallas as pl
from jax.experimental.pallas import tpu as pltpu
from jax.experimental.pallas import tpu_sc as plsc

N = 10000
E = 160000
C = 256
HALF = C // 2
EPS = 1e-5

NC = 2
NS = 16

ROW_BLK = 1000
GRID = N // ROW_BLK

EK = 125
EROWS = E // EK
DEG_RPW = EROWS // NS
AGG_RPT = EROWS // NS
AGG_BATCH = 16

NPAD = 10240
STRIPE = NPAD // NS
SEED_CHUNK = 32


def _sc_mesh():
    return plsc.VectorSubcoreMesh(
        core_axis_name="c", subcore_axis_name="s", num_cores=NC, num_subcores=NS
    )


def _tc1_body(x_ref, wt_ref, b_ref, h_ref, stats_ref):
    i = pl.program_id(0)
    h = jnp.dot(x_ref[...], wt_ref[...], preferred_element_type=jnp.float32)
    h = h + b_ref[...]
    h_ref[...] = h
    s = jnp.sum(h, axis=0, keepdims=True)
    ss = jnp.sum(h * h, axis=0, keepdims=True)
    blk = jnp.concatenate([s, ss, jnp.zeros((6, C), jnp.float32)], axis=0)
    prev = jnp.where(i == 0, jnp.zeros_like(blk), stats_ref[...])
    stats_ref[...] = prev + blk


def _tc1(x, wt, b2):
    return pl.pallas_call(
        _tc1_body,
        grid=(GRID,),
        in_specs=[
            pl.BlockSpec((ROW_BLK, C), lambda i: (i, 0)),
            pl.BlockSpec((C, C), lambda i: (0, 0)),
            pl.BlockSpec((1, C), lambda i: (0, 0)),
        ],
        out_specs=[
            pl.BlockSpec((ROW_BLK, C), lambda i: (i, 0)),
            pl.BlockSpec((8, C), lambda i: (0, 0)),
        ],
        out_shape=[
            jax.ShapeDtypeStruct((N, C), jnp.float32),
            jax.ShapeDtypeStruct((8, C), jnp.float32),
        ],
    )(x, wt, b2)


def _tc2_body(h_ref, stats_ref, p_ref, g_ref, be_ref, hs2_ref):
    inv_n = 1.0 / N
    mean = stats_ref[0:1, :] * inv_n
    var = stats_ref[1:2, :] * inv_n - mean * mean
    a = g_ref[...] * lax.rsqrt(var + EPS)
    cc = be_ref[...] - a * mean
    dinv = lax.rsqrt(p_ref[...] + 1.0)
    hs = (h_ref[...] * a + cc) * dinv
    hs2_ref[...] = jnp.stack([hs[:, :HALF], hs[:, HALF:]], axis=0)


def _tc2(h, stats, p, g2, be2):
    return pl.pallas_call(
        _tc2_body,
        grid=(GRID,),
        in_specs=[
            pl.BlockSpec((ROW_BLK, C), lambda i: (i, 0)),
            pl.BlockSpec((8, C), lambda i: (0, 0)),
            pl.BlockSpec((ROW_BLK, 1), lambda i: (i, 0)),
            pl.BlockSpec((1, C), lambda i: (0, 0)),
            pl.BlockSpec((1, C), lambda i: (0, 0)),
        ],
        out_specs=pl.BlockSpec((NC, ROW_BLK, HALF), lambda i: (0, i, 0)),
        out_shape=jax.ShapeDtypeStruct((NC, NPAD, HALF), jnp.float32),
    )(h, stats, p, g2, be2)


def _deg_body(edge_hbm, deg_hbm, idx_v, ones_v, stage_v, deg_sh):
    c = lax.axis_index("c")
    s = lax.axis_index("s")
    dst_hbm = edge_hbm.at[1]

    one16 = jnp.ones((16,), jnp.float32)
    for k in range(8):
        ones_v[pl.ds(k * 16, 16)] = one16
    z16 = jnp.zeros((16,), jnp.float32)
    for k in range(STRIPE // 16):
        stage_v[pl.ds(k * 16, 16)] = z16

    off = pl.multiple_of(s * STRIPE, 8)
    pltpu.sync_copy(stage_v, deg_sh.at[pl.ds(off, STRIPE)])
    plsc.subcore_barrier()

    woff = pl.multiple_of(s * DEG_RPW, 8)
    pltpu.sync_copy(dst_hbm.at[pl.ds(woff, DEG_RPW)], idx_v)

    def body(j, carry):
        pltpu.sync_copy(ones_v.at[pl.ds(0, EK)],
                        deg_sh.at[idx_v.at[j]], add=True)
        return carry

    lax.fori_loop(0, DEG_RPW, body, 0)
    plsc.subcore_barrier()

    @pl.when(c == 0)
    def _():
        pltpu.sync_copy(deg_sh.at[pl.ds(off, STRIPE)], stage_v)
        pltpu.sync_copy(stage_v, deg_hbm.at[pl.ds(off, STRIPE)])


def _sc_degree(edge2):
    return pl.kernel(
        _deg_body,
        out_type=jax.ShapeDtypeStruct((NPAD,), jnp.float32),
        mesh=_sc_mesh(),
        scratch_types=[
            pltpu.VMEM((DEG_RPW, EK), jnp.int32),
            pltpu.VMEM((128,), jnp.float32),
            pltpu.VMEM((STRIPE,), jnp.float32),
            pltpu.VMEM_SHARED((NPAD,), jnp.float32),
        ],
    )(edge2)


def _agg_body(hs2_hbm, edge_hbm, agg2_hbm,
              idx_s0, idx_s1, idx_d0, idx_d1, rows0, rows1, stage, agg_sh,
              gsem0, gsem1, ssem0, ssem1):
    c = lax.axis_index("c")
    s = lax.axis_index("s")
    off = pl.multiple_of(s * STRIPE, 8)
    hs_ref = hs2_hbm.at[c]
    out_ref = agg2_hbm.at[c]
    src_hbm = edge_hbm.at[0]
    dst_hbm = edge_hbm.at[1]
    idx_s = (idx_s0, idx_s1)
    idx_d = (idx_d0, idx_d1)
    rows = (rows0, rows1)
    gsem = (gsem0, gsem1)
    ssem = (ssem0, ssem1)

    boff0 = pl.multiple_of(s * AGG_RPT, 8)
    pltpu.sync_copy(src_hbm.at[pl.ds(boff0, AGG_BATCH)], idx_s[0])
    pltpu.sync_copy(dst_hbm.at[pl.ds(boff0, AGG_BATCH)], idx_d[0])
    gd = [None, None]
    sd = [None, None]
    gd[0] = pltpu.async_copy(hs_ref.at[idx_s[0].at[0]], rows[0], gsem[0])
    gd[1] = pltpu.async_copy(hs_ref.at[idx_s[0].at[1]], rows[1], gsem[1])
    for k in range(STRIPE // SEED_CHUNK):
        koff = pl.multiple_of(off + k * SEED_CHUNK, 8)
        pltpu.sync_copy(hs_ref.at[pl.ds(koff, SEED_CHUNK)], stage)
        pltpu.sync_copy(stage, agg_sh.at[pl.ds(koff, SEED_CHUNK)])
    plsc.subcore_barrier()

    for j in range(AGG_RPT):
        slot = j % 2
        b, r = divmod(j, AGG_BATCH)
        gd[slot].wait()
        sd[slot] = pltpu.async_copy(
            rows[slot], agg_sh.at[idx_d[b % 2].at[r]], ssem[slot],
            add=True)
        nj = j + 2
        if nj < AGG_RPT:
            nb, nr = divmod(nj, AGG_BATCH)
            if nr == 0:
                boff = pl.multiple_of(s * AGG_RPT + nb * AGG_BATCH, 8)
                pltpu.sync_copy(src_hbm.at[pl.ds(boff, AGG_BATCH)],
                                idx_s[nb % 2])
                pltpu.sync_copy(dst_hbm.at[pl.ds(boff, AGG_BATCH)],
                                idx_d[nb % 2])
            sd[slot].wait()
            gd[slot] = pltpu.async_copy(
                hs_ref.at[idx_s[nb % 2].at[nr]], rows[slot], gsem[slot])
    sd[(AGG_RPT - 1) % 2].wait()
    sd[AGG_RPT % 2].wait()
    plsc.subcore_barrier()

    for k in range(STRIPE // SEED_CHUNK):
        koff = pl.multiple_of(off + k * SEED_CHUNK, 8)
        pltpu.sync_copy(agg_sh.at[pl.ds(koff, SEED_CHUNK)], stage)
        pltpu.sync_copy(stage, out_ref.at[pl.ds(koff, SEED_CHUNK)])


def _sc_agg(hs2, edge2):
    return pl.kernel(
        _agg_body,
        out_type=jax.ShapeDtypeStruct((NC, NPAD, HALF), jnp.float32),
        mesh=_sc_mesh(),
        scratch_types=[
            pltpu.VMEM((AGG_BATCH, EK), jnp.int32),
            pltpu.VMEM((AGG_BATCH, EK), jnp.int32),
            pltpu.VMEM((AGG_BATCH, EK), jnp.int32),
            pltpu.VMEM((AGG_BATCH, EK), jnp.int32),
            pltpu.VMEM((EK, HALF), jnp.float32),
            pltpu.VMEM((EK, HALF), jnp.float32),
            pltpu.VMEM((SEED_CHUNK, HALF), jnp.float32),
            pltpu.VMEM_SHARED((NPAD, HALF), jnp.float32),
            pltpu.SemaphoreType.DMA,
            pltpu.SemaphoreType.DMA,
            pltpu.SemaphoreType.DMA,
            pltpu.SemaphoreType.DMA,
        ],
    )(hs2, edge2)


def _tc3_body(a2_ref, p_ref, out_ref):
    dinv = lax.rsqrt(p_ref[...] + 1.0)
    a2 = a2_ref[...]
    o = jnp.concatenate([a2[0], a2[1]], axis=1) * dinv
    out_ref[...] = jnp.maximum(o, 0.0)


def _tc3(agg2, p):
    return pl.pallas_call(
        _tc3_body,
        grid=(GRID,),
        in_specs=[
            pl.BlockSpec((NC, ROW_BLK, HALF), lambda i: (0, i, 0)),
            pl.BlockSpec((ROW_BLK, 1), lambda i: (i, 0)),
        ],
        out_specs=pl.BlockSpec((ROW_BLK, C), lambda i: (i, 0)),
        out_shape=jax.ShapeDtypeStruct((N, C), jnp.float32),
    )(agg2, p)


def kernel(X, edge_index, W, b, gamma, beta):
    edge2 = edge_index.reshape(2, EROWS, EK)
    wt = W.T
    b2 = b.reshape(1, C)
    g2 = gamma.reshape(1, C)
    be2 = beta.reshape(1, C)

    deg = _sc_degree(edge2)
    h, stats = _tc1(X, wt, b2)
    p = deg.reshape(NPAD, 1)
    hs2 = _tc2(h, stats, p, g2, be2)
    agg2 = _sc_agg(hs2, edge2)
    return _tc3(agg2, p)


# speedup vs baseline: 23.3434x; 1.0386x over previous
import jax
import jax.numpy as jnp
from jax import lax
from jax.experimental import pallas as pl
from jax.experimental.pallas import tpu as pltpu
from jax.experimental.pallas import tpu_sc as plsc

N = 10000
E = 160000
C = 256
HALF = C // 2
EPS = 1e-5

NC = 2
NS = 16

ROW_BLK = 1000
GRID = N // ROW_BLK

EK = 125
EROWS = E // EK
DEG_RPW = EROWS // NS
AGG_RPT = EROWS // NS
AGG_BATCH = 16

NPAD = 10240
STRIPE = NPAD // NS
SEED_CHUNK = 32


def _sc_mesh():
    return plsc.VectorSubcoreMesh(
        core_axis_name="c", subcore_axis_name="s", num_cores=NC, num_subcores=NS
    )


def _tc1_body(x_ref, wt_ref, b_ref, h_ref, stats_ref):
    i = pl.program_id(0)
    h = jnp.dot(x_ref[...], wt_ref[...], preferred_element_type=jnp.float32)
    h = h + b_ref[...]
    h_ref[...] = h
    s = jnp.sum(h, axis=0, keepdims=True)
    ss = jnp.sum(h * h, axis=0, keepdims=True)
    blk = jnp.concatenate([s, ss, jnp.zeros((6, C), jnp.float32)], axis=0)
    prev = jnp.where(i == 0, jnp.zeros_like(blk), stats_ref[...])
    stats_ref[...] = prev + blk


def _tc1(x, wt, b2):
    return pl.pallas_call(
        _tc1_body,
        grid=(GRID,),
        in_specs=[
            pl.BlockSpec((ROW_BLK, C), lambda i: (i, 0)),
            pl.BlockSpec((C, C), lambda i: (0, 0)),
            pl.BlockSpec((1, C), lambda i: (0, 0)),
        ],
        out_specs=[
            pl.BlockSpec((ROW_BLK, C), lambda i: (i, 0)),
            pl.BlockSpec((8, C), lambda i: (0, 0)),
        ],
        out_shape=[
            jax.ShapeDtypeStruct((N, C), jnp.float32),
            jax.ShapeDtypeStruct((8, C), jnp.float32),
        ],
    )(x, wt, b2)


def _tc2_body(h_ref, stats_ref, p_ref, g_ref, be_ref, hs2_ref):
    inv_n = 1.0 / N
    mean = stats_ref[0:1, :] * inv_n
    var = stats_ref[1:2, :] * inv_n - mean * mean
    a = g_ref[...] * lax.rsqrt(var + EPS)
    cc = be_ref[...] - a * mean
    dinv = lax.rsqrt(p_ref[...] + 1.0)
    hs = (h_ref[...] * a + cc) * dinv
    hs2_ref[...] = jnp.stack([hs[:, :HALF], hs[:, HALF:]], axis=0)


def _tc2(h, stats, p, g2, be2):
    return pl.pallas_call(
        _tc2_body,
        grid=(GRID,),
        in_specs=[
            pl.BlockSpec((ROW_BLK, C), lambda i: (i, 0)),
            pl.BlockSpec((8, C), lambda i: (0, 0)),
            pl.BlockSpec((ROW_BLK, 1), lambda i: (i, 0)),
            pl.BlockSpec((1, C), lambda i: (0, 0)),
            pl.BlockSpec((1, C), lambda i: (0, 0)),
        ],
        out_specs=pl.BlockSpec((NC, ROW_BLK, HALF), lambda i: (0, i, 0)),
        out_shape=jax.ShapeDtypeStruct((NC, NPAD, HALF), jnp.float32),
    )(h, stats, p, g2, be2)


def _deg_body(edge_hbm, deg_hbm, idx_v, ones_v, stage_v, deg_sh):
    c = lax.axis_index("c")
    s = lax.axis_index("s")
    dst_hbm = edge_hbm.at[1]

    one16 = jnp.ones((16,), jnp.float32)
    for k in range(8):
        ones_v[pl.ds(k * 16, 16)] = one16
    z16 = jnp.zeros((16,), jnp.float32)
    for k in range(STRIPE // 16):
        stage_v[pl.ds(k * 16, 16)] = z16

    off = pl.multiple_of(s * STRIPE, 8)
    pltpu.sync_copy(stage_v, deg_sh.at[pl.ds(off, STRIPE)])
    plsc.subcore_barrier()

    woff = pl.multiple_of(s * DEG_RPW, 8)
    pltpu.sync_copy(dst_hbm.at[pl.ds(woff, DEG_RPW)], idx_v)

    def body(j, carry):
        pltpu.sync_copy(ones_v.at[pl.ds(0, EK)],
                        deg_sh.at[idx_v.at[j]], add=True)
        return carry

    lax.fori_loop(0, DEG_RPW, body, 0)
    plsc.subcore_barrier()

    @pl.when(c == 0)
    def _():
        pltpu.sync_copy(deg_sh.at[pl.ds(off, STRIPE)], stage_v)
        pltpu.sync_copy(stage_v, deg_hbm.at[pl.ds(off, STRIPE)])


def _sc_degree(edge2):
    return pl.kernel(
        _deg_body,
        out_type=jax.ShapeDtypeStruct((NPAD,), jnp.float32),
        mesh=_sc_mesh(),
        scratch_types=[
            pltpu.VMEM((DEG_RPW, EK), jnp.int32),
            pltpu.VMEM((128,), jnp.float32),
            pltpu.VMEM((STRIPE,), jnp.float32),
            pltpu.VMEM_SHARED((NPAD,), jnp.float32),
        ],
    )(edge2)


def _agg_body(hs2_hbm, edge_hbm, agg2_hbm,
              idx_s0, idx_s1, idx_d0, idx_d1, rows0, rows1, stage0, stage1,
              agg_sh, gsem0, gsem1, ssem0, ssem1):
    c = lax.axis_index("c")
    s = lax.axis_index("s")
    off = pl.multiple_of(s * STRIPE, 8)
    hs_ref = hs2_hbm.at[c]
    out_ref = agg2_hbm.at[c]
    src_hbm = edge_hbm.at[0]
    dst_hbm = edge_hbm.at[1]
    idx_s = (idx_s0, idx_s1)
    idx_d = (idx_d0, idx_d1)
    rows = (rows0, rows1)
    stage = (stage0, stage1)
    gsem = (gsem0, gsem1)
    ssem = (ssem0, ssem1)

    boff0 = pl.multiple_of(s * AGG_RPT, 8)
    pltpu.sync_copy(src_hbm.at[pl.ds(boff0, AGG_BATCH)], idx_s[0])
    pltpu.sync_copy(dst_hbm.at[pl.ds(boff0, AGG_BATCH)], idx_d[0])
    gd = [None, None]
    sd = [None, None]
    gd[0] = pltpu.async_copy(hs_ref.at[idx_s[0].at[0]], rows[0], gsem[0])
    gd[1] = pltpu.async_copy(hs_ref.at[idx_s[0].at[1]], rows[1], gsem[1])
    seed_d = [None, None]
    for k in range(STRIPE // SEED_CHUNK):
        sl = k % 2
        if seed_d[sl] is not None:
            seed_d[sl].wait()
        koff = pl.multiple_of(off + k * SEED_CHUNK, 8)
        pltpu.sync_copy(hs_ref.at[pl.ds(koff, SEED_CHUNK)], stage[sl])
        seed_d[sl] = pltpu.async_copy(
            stage[sl], agg_sh.at[pl.ds(koff, SEED_CHUNK)], ssem[sl])
    seed_d[0].wait()
    seed_d[1].wait()
    plsc.subcore_barrier()

    for j in range(AGG_RPT):
        slot = j % 2
        b, r = divmod(j, AGG_BATCH)
        gd[slot].wait()
        sd[slot] = pltpu.async_copy(
            rows[slot], agg_sh.at[idx_d[b % 2].at[r]], ssem[slot],
            add=True)
        nj = j + 2
        if nj < AGG_RPT:
            nb, nr = divmod(nj, AGG_BATCH)
            if nr == 0:
                boff = pl.multiple_of(s * AGG_RPT + nb * AGG_BATCH, 8)
                pltpu.sync_copy(src_hbm.at[pl.ds(boff, AGG_BATCH)],
                                idx_s[nb % 2])
                pltpu.sync_copy(dst_hbm.at[pl.ds(boff, AGG_BATCH)],
                                idx_d[nb % 2])
            sd[slot].wait()
            gd[slot] = pltpu.async_copy(
                hs_ref.at[idx_s[nb % 2].at[nr]], rows[slot], gsem[slot])
    sd[(AGG_RPT - 1) % 2].wait()
    sd[AGG_RPT % 2].wait()
    plsc.subcore_barrier()

    drain_d = [None, None]
    for k in range(STRIPE // SEED_CHUNK):
        sl = k % 2
        if drain_d[sl] is not None:
            drain_d[sl].wait()
        koff = pl.multiple_of(off + k * SEED_CHUNK, 8)
        pltpu.sync_copy(agg_sh.at[pl.ds(koff, SEED_CHUNK)], stage[sl])
        drain_d[sl] = pltpu.async_copy(
            stage[sl], out_ref.at[pl.ds(koff, SEED_CHUNK)], ssem[sl])
    drain_d[0].wait()
    drain_d[1].wait()


def _sc_agg(hs2, edge2):
    return pl.kernel(
        _agg_body,
        out_type=jax.ShapeDtypeStruct((NC, NPAD, HALF), jnp.float32),
        mesh=_sc_mesh(),
        scratch_types=[
            pltpu.VMEM((AGG_BATCH, EK), jnp.int32),
            pltpu.VMEM((AGG_BATCH, EK), jnp.int32),
            pltpu.VMEM((AGG_BATCH, EK), jnp.int32),
            pltpu.VMEM((AGG_BATCH, EK), jnp.int32),
            pltpu.VMEM((EK, HALF), jnp.float32),
            pltpu.VMEM((EK, HALF), jnp.float32),
            pltpu.VMEM((SEED_CHUNK, HALF), jnp.float32),
            pltpu.VMEM((SEED_CHUNK, HALF), jnp.float32),
            pltpu.VMEM_SHARED((NPAD, HALF), jnp.float32),
            pltpu.SemaphoreType.DMA,
            pltpu.SemaphoreType.DMA,
            pltpu.SemaphoreType.DMA,
            pltpu.SemaphoreType.DMA,
        ],
    )(hs2, edge2)


def _tc3_body(a2_ref, p_ref, out_ref):
    dinv = lax.rsqrt(p_ref[...] + 1.0)
    a2 = a2_ref[...]
    o = jnp.concatenate([a2[0], a2[1]], axis=1) * dinv
    out_ref[...] = jnp.maximum(o, 0.0)


def _tc3(agg2, p):
    return pl.pallas_call(
        _tc3_body,
        grid=(GRID,),
        in_specs=[
            pl.BlockSpec((NC, ROW_BLK, HALF), lambda i: (0, i, 0)),
            pl.BlockSpec((ROW_BLK, 1), lambda i: (i, 0)),
        ],
        out_specs=pl.BlockSpec((ROW_BLK, C), lambda i: (i, 0)),
        out_shape=jax.ShapeDtypeStruct((N, C), jnp.float32),
    )(agg2, p)


def kernel(X, edge_index, W, b, gamma, beta):
    edge2 = edge_index.reshape(2, EROWS, EK)
    wt = W.T
    b2 = b.reshape(1, C)
    g2 = gamma.reshape(1, C)
    be2 = beta.reshape(1, C)

    deg = _sc_degree(edge2)
    h, stats = _tc1(X, wt, b2)
    p = deg.reshape(NPAD, 1)
    hs2 = _tc2(h, stats, p, g2, be2)
    agg2 = _sc_agg(hs2, edge2)
    return _tc3(agg2, p)


# speedup vs baseline: 23.5320x; 1.0081x over previous
import jax
import jax.numpy as jnp
from jax import lax
from jax.experimental import pallas as pl
from jax.experimental.pallas import tpu as pltpu
from jax.experimental.pallas import tpu_sc as plsc

N = 10000
E = 160000
C = 256
HALF = C // 2
EPS = 1e-5

NC = 2
NS = 16

ROW_BLK = 1000
GRID = N // ROW_BLK

EK = 125
EROWS = E // EK
DEG_RPW = EROWS // NS
AGG_RPT = EROWS // NS
AGG_BATCH = 16

NPAD = 10240
STRIPE = NPAD // NS
SEED_CHUNK = 32


def _sc_mesh():
    return plsc.VectorSubcoreMesh(
        core_axis_name="c", subcore_axis_name="s", num_cores=NC, num_subcores=NS
    )


def _tc1_body(x_ref, w_ref, b_ref, h_ref, stats_ref):
    i = pl.program_id(0)
    h = lax.dot_general(
        x_ref[...], w_ref[...], (((1,), (1,)), ((), ())),
        preferred_element_type=jnp.float32)
    h = h + b_ref[...]
    h_ref[...] = h
    s = jnp.sum(h, axis=0, keepdims=True)
    ss = jnp.sum(h * h, axis=0, keepdims=True)
    blk = jnp.concatenate([s, ss, jnp.zeros((6, C), jnp.float32)], axis=0)
    prev = jnp.where(i == 0, jnp.zeros_like(blk), stats_ref[...])
    stats_ref[...] = prev + blk


def _tc1(x, wt, b2):
    return pl.pallas_call(
        _tc1_body,
        grid=(GRID,),
        in_specs=[
            pl.BlockSpec((ROW_BLK, C), lambda i: (i, 0)),
            pl.BlockSpec((C, C), lambda i: (0, 0)),
            pl.BlockSpec((1, C), lambda i: (0, 0)),
        ],
        out_specs=[
            pl.BlockSpec((ROW_BLK, C), lambda i: (i, 0)),
            pl.BlockSpec((8, C), lambda i: (0, 0)),
        ],
        out_shape=[
            jax.ShapeDtypeStruct((N, C), jnp.float32),
            jax.ShapeDtypeStruct((8, C), jnp.float32),
        ],
    )(x, wt, b2)


def _tc2_body(h_ref, stats_ref, p_ref, g_ref, be_ref, hs2_ref):
    inv_n = 1.0 / N
    mean = stats_ref[0:1, :] * inv_n
    var = stats_ref[1:2, :] * inv_n - mean * mean
    a = g_ref[...] * lax.rsqrt(var + EPS)
    cc = be_ref[...] - a * mean
    dinv = lax.rsqrt(p_ref[...] + 1.0)
    hs = (h_ref[...] * a + cc) * dinv
    hs2_ref[...] = jnp.stack([hs[:, :HALF], hs[:, HALF:]], axis=0)


def _tc2(h, stats, p, g2, be2):
    return pl.pallas_call(
        _tc2_body,
        grid=(GRID,),
        in_specs=[
            pl.BlockSpec((ROW_BLK, C), lambda i: (i, 0)),
            pl.BlockSpec((8, C), lambda i: (0, 0)),
            pl.BlockSpec((ROW_BLK, 1), lambda i: (i, 0)),
            pl.BlockSpec((1, C), lambda i: (0, 0)),
            pl.BlockSpec((1, C), lambda i: (0, 0)),
        ],
        out_specs=pl.BlockSpec((NC, ROW_BLK, HALF), lambda i: (0, i, 0)),
        out_shape=jax.ShapeDtypeStruct((NC, NPAD, HALF), jnp.float32),
    )(h, stats, p, g2, be2)


def _deg_body(edge_hbm, deg_hbm, idx_v, ones_v, stage_v, deg_sh):
    c = lax.axis_index("c")
    s = lax.axis_index("s")
    dst_hbm = edge_hbm.at[1]

    one16 = jnp.ones((16,), jnp.float32)
    for k in range(8):
        ones_v[pl.ds(k * 16, 16)] = one16
    z16 = jnp.zeros((16,), jnp.float32)
    for k in range(STRIPE // 16):
        stage_v[pl.ds(k * 16, 16)] = z16

    off = pl.multiple_of(s * STRIPE, 8)
    pltpu.sync_copy(stage_v, deg_sh.at[pl.ds(off, STRIPE)])
    plsc.subcore_barrier()

    woff = pl.multiple_of(s * DEG_RPW, 8)
    pltpu.sync_copy(dst_hbm.at[pl.ds(woff, DEG_RPW)], idx_v)

    def body(j, carry):
        pltpu.sync_copy(ones_v.at[pl.ds(0, EK)],
                        deg_sh.at[idx_v.at[j]], add=True)
        return carry

    lax.fori_loop(0, DEG_RPW, body, 0)
    plsc.subcore_barrier()

    @pl.when(c == 0)
    def _():
        pltpu.sync_copy(deg_sh.at[pl.ds(off, STRIPE)], stage_v)
        pltpu.sync_copy(stage_v, deg_hbm.at[pl.ds(off, STRIPE)])


def _sc_degree(edge2):
    return pl.kernel(
        _deg_body,
        out_type=jax.ShapeDtypeStruct((NPAD,), jnp.float32),
        mesh=_sc_mesh(),
        scratch_types=[
            pltpu.VMEM((DEG_RPW, EK), jnp.int32),
            pltpu.VMEM((128,), jnp.float32),
            pltpu.VMEM((STRIPE,), jnp.float32),
            pltpu.VMEM_SHARED((NPAD,), jnp.float32),
        ],
    )(edge2)


def _agg_body(hs2_hbm, edge_hbm, agg2_hbm,
              idx_s0, idx_s1, idx_d0, idx_d1, rows0, rows1, stage0, stage1,
              agg_sh, gsem0, gsem1, ssem0, ssem1):
    c = lax.axis_index("c")
    s = lax.axis_index("s")
    off = pl.multiple_of(s * STRIPE, 8)
    hs_ref = hs2_hbm.at[c]
    out_ref = agg2_hbm.at[c]
    src_hbm = edge_hbm.at[0]
    dst_hbm = edge_hbm.at[1]
    idx_s = (idx_s0, idx_s1)
    idx_d = (idx_d0, idx_d1)
    rows = (rows0, rows1)
    stage = (stage0, stage1)
    gsem = (gsem0, gsem1)
    ssem = (ssem0, ssem1)

    boff0 = pl.multiple_of(s * AGG_RPT, 8)
    pltpu.sync_copy(src_hbm.at[pl.ds(boff0, AGG_BATCH)], idx_s[0])
    pltpu.sync_copy(dst_hbm.at[pl.ds(boff0, AGG_BATCH)], idx_d[0])
    gd = [None, None]
    sd = [None, None]
    gd[0] = pltpu.async_copy(hs_ref.at[idx_s[0].at[0]], rows[0], gsem[0])
    gd[1] = pltpu.async_copy(hs_ref.at[idx_s[0].at[1]], rows[1], gsem[1])
    seed_d = [None, None]
    for k in range(STRIPE // SEED_CHUNK):
        sl = k % 2
        if seed_d[sl] is not None:
            seed_d[sl].wait()
        koff = pl.multiple_of(off + k * SEED_CHUNK, 8)
        pltpu.sync_copy(hs_ref.at[pl.ds(koff, SEED_CHUNK)], stage[sl])
        seed_d[sl] = pltpu.async_copy(
            stage[sl], agg_sh.at[pl.ds(koff, SEED_CHUNK)], ssem[sl])
    seed_d[0].wait()
    seed_d[1].wait()
    plsc.subcore_barrier()

    for j in range(AGG_RPT):
        slot = j % 2
        b, r = divmod(j, AGG_BATCH)
        gd[slot].wait()
        sd[slot] = pltpu.async_copy(
            rows[slot], agg_sh.at[idx_d[b % 2].at[r]], ssem[slot],
            add=True)
        nj = j + 2
        if nj < AGG_RPT:
            nb, nr = divmod(nj, AGG_BATCH)
            if nr == 0:
                boff = pl.multiple_of(s * AGG_RPT + nb * AGG_BATCH, 8)
                pltpu.sync_copy(src_hbm.at[pl.ds(boff, AGG_BATCH)],
                                idx_s[nb % 2])
                pltpu.sync_copy(dst_hbm.at[pl.ds(boff, AGG_BATCH)],
                                idx_d[nb % 2])
            sd[slot].wait()
            gd[slot] = pltpu.async_copy(
                hs_ref.at[idx_s[nb % 2].at[nr]], rows[slot], gsem[slot])
    sd[(AGG_RPT - 1) % 2].wait()
    sd[AGG_RPT % 2].wait()
    plsc.subcore_barrier()

    drain_d = [None, None]
    for k in range(STRIPE // SEED_CHUNK):
        sl = k % 2
        if drain_d[sl] is not None:
            drain_d[sl].wait()
        koff = pl.multiple_of(off + k * SEED_CHUNK, 8)
        pltpu.sync_copy(agg_sh.at[pl.ds(koff, SEED_CHUNK)], stage[sl])
        drain_d[sl] = pltpu.async_copy(
            stage[sl], out_ref.at[pl.ds(koff, SEED_CHUNK)], ssem[sl])
    drain_d[0].wait()
    drain_d[1].wait()


def _sc_agg(hs2, edge2):
    return pl.kernel(
        _agg_body,
        out_type=jax.ShapeDtypeStruct((NC, NPAD, HALF), jnp.float32),
        mesh=_sc_mesh(),
        scratch_types=[
            pltpu.VMEM((AGG_BATCH, EK), jnp.int32),
            pltpu.VMEM((AGG_BATCH, EK), jnp.int32),
            pltpu.VMEM((AGG_BATCH, EK), jnp.int32),
            pltpu.VMEM((AGG_BATCH, EK), jnp.int32),
            pltpu.VMEM((EK, HALF), jnp.float32),
            pltpu.VMEM((EK, HALF), jnp.float32),
            pltpu.VMEM((SEED_CHUNK, HALF), jnp.float32),
            pltpu.VMEM((SEED_CHUNK, HALF), jnp.float32),
            pltpu.VMEM_SHARED((NPAD, HALF), jnp.float32),
            pltpu.SemaphoreType.DMA,
            pltpu.SemaphoreType.DMA,
            pltpu.SemaphoreType.DMA,
            pltpu.SemaphoreType.DMA,
        ],
    )(hs2, edge2)


def _tc3_body(a2_ref, p_ref, out_ref):
    dinv = lax.rsqrt(p_ref[...] + 1.0)
    a2 = a2_ref[...]
    o = jnp.concatenate([a2[0], a2[1]], axis=1) * dinv
    out_ref[...] = jnp.maximum(o, 0.0)


def _tc3(agg2, p):
    return pl.pallas_call(
        _tc3_body,
        grid=(GRID,),
        in_specs=[
            pl.BlockSpec((NC, ROW_BLK, HALF), lambda i: (0, i, 0)),
            pl.BlockSpec((ROW_BLK, 1), lambda i: (i, 0)),
        ],
        out_specs=pl.BlockSpec((ROW_BLK, C), lambda i: (i, 0)),
        out_shape=jax.ShapeDtypeStruct((N, C), jnp.float32),
    )(agg2, p)


def kernel(X, edge_index, W, b, gamma, beta):
    edge2 = edge_index.reshape(2, EROWS, EK)
    b2 = b.reshape(1, C)
    g2 = gamma.reshape(1, C)
    be2 = beta.reshape(1, C)

    deg = _sc_degree(edge2)
    h, stats = _tc1(X, W, b2)
    p = deg.reshape(NPAD, 1)
    hs2 = _tc2(h, stats, p, g2, be2)
    agg2 = _sc_agg(hs2, edge2)
    return _tc3(agg2, p)
